# baseline (speedup 1.0000x reference)
import functools

import jax
import jax.numpy as jnp
from jax import lax
from jax.experimental import pallas as pl
from jax.experimental.pallas import tpu as pltpu
from jax.experimental.pallas import tpu_sc as plsc

B, C, H, W = 32, 64, 128, 128
HW = H * W
K = 500
KP = 512
NC, NS, L = 2, 16, 16
NCHUNK = KP // L
GW = 128
NG = KP // GW


def _sc_body(outs_hbm, cind_hbm, tgt_hbm, nbs_hbm, out_hbm,
             cind_v, nb_v, idx0, idx1, pred0, pred1, tgt0, tgt1, res_v,
             sem0, sem1, ts0, ts1):
    b = lax.axis_index("s") * NC + lax.axis_index("c")
    pltpu.sync_copy(cind_hbm.at[b], cind_v)
    pltpu.sync_copy(nbs_hbm.at[b], nb_v)
    nb = nb_v[pl.ds(0, L)][0]

    iota = lax.iota(jnp.int32, L)
    bufs = ((idx0, pred0, tgt0, sem0, ts0), (idx1, pred1, tgt1, sem1, ts1))

    def fire(c, buf):
        idx_v, pred_v, tgt_v, sem, tsem = buf
        base = (b * C + c) * HW
        pltpu.make_async_copy(tgt_hbm.at[b, c], tgt_v, tsem).start()
        for r in range(NG):
            @pl.when(r * GW < nb)
            def _():
                for j in range(GW // L):
                    o = r * GW + j * L
                    idx_v[pl.ds(o, L)] = cind_v[pl.ds(o, L)] + base
                pltpu.make_async_copy(
                    outs_hbm.at[idx_v.at[pl.ds(r * GW, GW)]],
                    pred_v.at[pl.ds(r * GW, GW)], sem).start()

    def drain_accum(c, buf, acc):
        idx_v, pred_v, tgt_v, sem, tsem = buf
        pltpu.make_async_copy(tgt_hbm.at[b, c], tgt_v, tsem).wait()
        for r in range(NG):
            @pl.when(r * GW < nb)
            def _():
                pltpu.make_async_copy(
                    outs_hbm.at[idx_v.at[pl.ds(r * GW, GW)]],
                    pred_v.at[pl.ds(r * GW, GW)], sem).wait()
        for j in range(NCHUNK):
            pv = pred_v[pl.ds(j * L, L)]
            tv = tgt_v[pl.ds(j * L, L)]
            valid = (iota + (j * L)) < nb
            acc = acc + jnp.where(valid, jnp.abs(pv - tv), 0.0)
        return acc

    zero = jnp.zeros((L,), jnp.float32)

    fire(0, bufs[0])

    def pair_step(i, acc):
        c0 = 2 * i
        fire(c0 + 1, bufs[1])
        acc = drain_accum(c0, bufs[0], acc)

        @pl.when(c0 + 2 < C)
        def _():
            fire(c0 + 2, bufs[0])

        return drain_accum(c0 + 1, bufs[1], acc)

    acc = lax.fori_loop(0, C // 2, pair_step, zero)
    res_v[0, :] = acc
    res_v[1, :] = jnp.where(iota == 0, nb.astype(jnp.float32), 0.0)
    pltpu.sync_copy(res_v, out_hbm.at[b])


@jax.jit
def kernel(outputs_key, targets_mask_key, targets_ind_key, targets_key):
    outs_flat = outputs_key.reshape(B * C * HW)

    mask = targets_mask_key
    pos = jnp.cumsum(mask, axis=1) - 1
    safe_pos = jnp.where(mask > 0, pos, KP - 1)
    rows = jnp.arange(B, dtype=jnp.int32)[:, None]
    karr = jnp.broadcast_to(jnp.arange(K, dtype=jnp.int32), (B, K))
    cidx = jnp.zeros((B, KP), jnp.int32).at[rows, safe_pos].set(karr)
    cind = jnp.take_along_axis(targets_ind_key, cidx, axis=1)
    tgt_c = jnp.take_along_axis(jnp.transpose(targets_key, (0, 2, 1)),
                                cidx[:, None, :], axis=2)
    nbs = jnp.broadcast_to(jnp.sum(mask, axis=1, dtype=jnp.int32)[:, None],
                           (B, L))

    mesh = plsc.VectorSubcoreMesh(core_axis_name="c", subcore_axis_name="s")
    f = pl.kernel(
        _sc_body,
        out_type=jax.ShapeDtypeStruct((B, 2, L), jnp.float32),
        mesh=mesh,
        scratch_types=[
            pltpu.VMEM((KP,), jnp.int32),
            pltpu.VMEM((L,), jnp.int32),
            pltpu.VMEM((KP,), jnp.int32),
            pltpu.VMEM((KP,), jnp.int32),
            pltpu.VMEM((KP,), jnp.float32),
            pltpu.VMEM((KP,), jnp.float32),
            pltpu.VMEM((KP,), jnp.float32),
            pltpu.VMEM((KP,), jnp.float32),
            pltpu.VMEM((2, L), jnp.float32),
            pltpu.SemaphoreType.DMA,
            pltpu.SemaphoreType.DMA,
            pltpu.SemaphoreType.DMA,
            pltpu.SemaphoreType.DMA,
        ],
    )
    part = f(outs_flat, cind, tgt_c, nbs)
    num = jnp.sum(part[:, 0, :])
    cnt = jnp.sum(part[:, 1, :])
    loss = num / (B * K * C)
    return loss / (C * cnt + 0.0001)


# speedup vs baseline: 2.9310x; 2.9310x over previous
import functools

import jax
import jax.numpy as jnp
from jax import lax
from jax.experimental import pallas as pl
from jax.experimental.pallas import tpu as pltpu
from jax.experimental.pallas import tpu_sc as plsc

B, C, H, W = 32, 64, 128, 128
HW = H * W
K = 500
KP = 512
NC, NS, L = 2, 16, 16
NCHUNK = KP // L
NBUF = 4


def _sc_body(outs_hbm, ind_hbm, mask_hbm, tgt_hbm, out_hbm,
             ind_v, mask_v,
             idxa, idxb, idxc, idxd, preda, predb, predc, predd,
             tgta, tgtb, tgtc, tgtd, res_v,
             sema, semb, semc, semd, tsa, tsb, tsc, tsd):
    b = lax.axis_index("s") * NC + lax.axis_index("c")
    pltpu.sync_copy(ind_hbm.at[b], ind_v)
    pltpu.sync_copy(mask_hbm.at[b], mask_v)

    idx_b = (idxa, idxb, idxc, idxd)
    pred_b = (preda, predb, predc, predd)
    tgt_b = (tgta, tgtb, tgtc, tgtd)
    sems = (sema, semb, semc, semd)
    tsems = (tsa, tsb, tsc, tsd)

    def fire(c, p):
        idx_v = idx_b[p]
        base = (b * C + c) * HW
        for j in range(NCHUNK):
            idx_v[pl.ds(j * L, L)] = ind_v[pl.ds(j * L, L)] + base
        pltpu.make_async_copy(tgt_hbm.at[b, c], tgt_b[p], tsems[p]).start()
        pltpu.make_async_copy(outs_hbm.at[idx_v], pred_b[p], sems[p]).start()

    def drain_accum(c, p, acc):
        pltpu.make_async_copy(tgt_hbm.at[b, c], tgt_b[p], tsems[p]).wait()
        pltpu.make_async_copy(outs_hbm.at[idx_b[p]], pred_b[p],
                              sems[p]).wait()
        pred_v = pred_b[p]
        tgt_v = tgt_b[p]
        for j in range(NCHUNK):
            pv = pred_v[pl.ds(j * L, L)]
            tv = tgt_v[pl.ds(j * L, L)]
            mv = mask_v[pl.ds(j * L, L)]
            acc = acc + jnp.where(mv > 0, jnp.abs(pv - tv), 0.0)
        return acc

    zero = jnp.zeros((L,), jnp.float32)
    cnt = zero
    for j in range(NCHUNK):
        cnt = cnt + mask_v[pl.ds(j * L, L)].astype(jnp.float32)

    for p in range(NBUF - 1):
        fire(p, p)

    def group_step(i, acc):
        c0 = NBUF * i
        for p in range(NBUF):
            nxt = c0 + p + NBUF - 1

            @pl.when(nxt < C)
            def _():
                fire(nxt, (p + NBUF - 1) % NBUF)

            acc = drain_accum(c0 + p, p, acc)
        return acc

    acc = lax.fori_loop(0, C // NBUF, group_step, zero)
    res_v[0, :] = acc
    res_v[1, :] = cnt
    pltpu.sync_copy(res_v, out_hbm.at[b])


@jax.jit
def kernel(outputs_key, targets_mask_key, targets_ind_key, targets_key):
    outs_flat = outputs_key.reshape(B * C * HW)
    ind_p = jnp.pad(targets_ind_key, ((0, 0), (0, KP - K)))
    mask_p = jnp.pad(targets_mask_key, ((0, 0), (0, KP - K)))
    tgt_t = jnp.pad(jnp.transpose(targets_key, (0, 2, 1)),
                    ((0, 0), (0, 0), (0, KP - K)))

    mesh = plsc.VectorSubcoreMesh(core_axis_name="c", subcore_axis_name="s")
    f = pl.kernel(
        _sc_body,
        out_type=jax.ShapeDtypeStruct((B, 2, L), jnp.float32),
        mesh=mesh,
        scratch_types=[
            pltpu.VMEM((KP,), jnp.int32),
            pltpu.VMEM((KP,), jnp.int32),
        ] + [pltpu.VMEM((KP,), jnp.int32)] * NBUF
          + [pltpu.VMEM((KP,), jnp.float32)] * NBUF
          + [pltpu.VMEM((KP,), jnp.float32)] * NBUF
          + [pltpu.VMEM((2, L), jnp.float32)]
          + [pltpu.SemaphoreType.DMA] * (2 * NBUF),
    )
    part = f(outs_flat, ind_p, mask_p, tgt_t)
    num = jnp.sum(part[:, 0, :])
    cnt = jnp.sum(part[:, 1, :])
    loss = num / (B * K * C)
    return loss / (C * cnt + 0.0001)
